# baseline (speedup 1.0000x reference)
import functools

import jax
import jax.numpy as jnp
from jax import lax
from jax.experimental import pallas as pl
from jax.experimental.pallas import tpu as pltpu
from jax.experimental.pallas import tpu_sc as plsc

N, S, A = 1024, 200, 3
E = 128
NROWS = N * S
NC, NSUB = 2, 16
NW = NC * NSUB
ROWS_PER_W = NROWS // NW
G = 128
NG = ROWS_PER_W // G


def _sc_body(t0, t1, t2, idx_hbm, out_hbm, idxv, didx, buf, sem0, sem1):
    c = lax.axis_index("c")
    s = lax.axis_index("s")
    wid = s * NC + c
    pltpu.sync_copy(idx_hbm.at[wid], idxv)
    tabs = (t0, t1, t2)
    sems = (sem0, sem1)
    i3 = lax.iota(jnp.int32, 16) * A

    def zero(p):
        z = jnp.zeros((16,), jnp.float32)

        def row(r, carry):
            for cc in range(E // 16):
                buf[p, r, pl.ds(cc * 16, 16)] = z
            return carry

        lax.fori_loop(0, G, row, 0)

    def deint(g, p):
        gbase = g * (G * A)
        for a in range(A):
            for ch in range(G // 16):
                v = plsc.load_gather(idxv, [i3 + (gbase + ch * 16 * A + a)])
                didx[pl.ds((p * A + a) * G + ch * 16, 16)] = v

    def issue(g, p):
        for a in range(A):
            pltpu.async_copy(
                tabs[a].at[didx.at[pl.ds((p * A + a) * G, G)]], buf.at[p], sems[p], add=True
            )

    def wait(g, p):
        for a in range(A):
            pltpu.make_async_copy(
                tabs[a].at[didx.at[pl.ds((p * A + a) * G, G)]], buf.at[p], sems[p]
            ).wait()

    def out(g, p):
        base = (wid * NG + g) * G
        pltpu.sync_copy(buf.at[p], out_hbm.at[pl.ds(base, G)])

    zero(0)
    zero(1)
    deint(0, 0)
    issue(0, 0)

    def pair(i, carry):
        g = 2 * i
        deint(g + 1, 1)
        issue(g + 1, 1)
        wait(g, 0)
        out(g, 0)
        zero(0)
        deint(g + 2, 0)
        issue(g + 2, 0)
        wait(g + 1, 1)
        out(g + 1, 1)
        zero(1)
        return carry

    lax.fori_loop(0, NG // 2 - 1, pair, 0)
    deint(NG - 1, 1)
    issue(NG - 1, 1)
    wait(NG - 2, 0)
    out(NG - 2, 0)
    wait(NG - 1, 1)
    out(NG - 1, 1)


_mesh = plsc.VectorSubcoreMesh(
    core_axis_name="c", subcore_axis_name="s", num_cores=NC, num_subcores=NSUB
)

_call = functools.partial(
    pl.kernel,
    out_type=jax.ShapeDtypeStruct((NROWS, E), jnp.float32),
    mesh=_mesh,
    compiler_params=pltpu.CompilerParams(needs_layout_passes=False),
    scratch_types=[
        pltpu.VMEM((NG * G * A,), jnp.int32),
        pltpu.VMEM((2 * A * G,), jnp.int32),
        pltpu.VMEM((2, G, E), jnp.float32),
        pltpu.SemaphoreType.DMA,
        pltpu.SemaphoreType.DMA,
    ],
)(_sc_body)


def kernel(position, table0, table1, table2):
    idx = position.reshape(NW, NG * G * A)
    out = _call(table0, table1, table2, idx)
    return out.reshape(N, S, E)


# speedup vs baseline: 2.1394x; 2.1394x over previous
import functools

import jax
import jax.numpy as jnp
from jax import lax
from jax.experimental import pallas as pl
from jax.experimental.pallas import tpu as pltpu
from jax.experimental.pallas import tpu_sc as plsc

N, S, A = 1024, 200, 3
E = 128
NROWS = N * S
NC, NSUB = 2, 16
NW = NC * NSUB
ROWS_PER_W = NROWS // NW
GSUB = 128
KSUB = 1
G = GSUB * KSUB
NG = ROWS_PER_W // G


def _sc_body(t0, t1, t2, idx_hbm, out_hbm, ts0, ts1, ts2, idxv, buf, sem0, sem1):
    c = lax.axis_index("c")
    s = lax.axis_index("s")
    wid = s * NC + c
    @pl.when(s == 0)
    def _stage():
        pltpu.sync_copy(t0, ts0)
        pltpu.sync_copy(t1, ts1)
        pltpu.sync_copy(t2, ts2)

    plsc.subcore_barrier()
    pltpu.sync_copy(idx_hbm.at[wid], idxv)
    tabs = (ts0, ts1, ts2)
    sems = (sem0, sem1)

    def zero(p):
        z = jnp.zeros((16,), jnp.float32)

        def row(r, carry):
            for cc in range(E // 16):
                buf[p, r, pl.ds(cc * 16, 16)] = z
            return carry

        lax.fori_loop(0, G, row, 0)

    def issue(g, p):
        for a in range(A):
            for k in range(KSUB):
                pltpu.async_copy(
                    tabs[a].at[idxv.at[a, g, k]],
                    buf.at[p, pl.ds(k * GSUB, GSUB)],
                    sems[p],
                    add=True,
                )

    def wait(g, p):
        for a in range(A):
            for k in range(KSUB):
                pltpu.make_async_copy(
                    tabs[a].at[idxv.at[a, g, k]],
                    buf.at[p, pl.ds(k * GSUB, GSUB)],
                    sems[p],
                ).wait()

    def out(g, p):
        base = (wid * NG + g) * G
        pltpu.sync_copy(buf.at[p], out_hbm.at[pl.ds(base, G)])

    zero(0)
    zero(1)
    issue(0, 0)

    def pair(i, carry):
        g = 2 * i
        issue(g + 1, 1)
        wait(g, 0)
        out(g, 0)
        zero(0)
        issue(g + 2, 0)
        wait(g + 1, 1)
        out(g + 1, 1)
        zero(1)
        return carry

    lax.fori_loop(0, NG // 2 - 1, pair, 0)
    issue(NG - 1, 1)
    wait(NG - 2, 0)
    out(NG - 2, 0)
    wait(NG - 1, 1)
    out(NG - 1, 1)


_mesh = plsc.VectorSubcoreMesh(
    core_axis_name="c", subcore_axis_name="s", num_cores=NC, num_subcores=NSUB
)

_call = functools.partial(
    pl.kernel,
    out_type=jax.ShapeDtypeStruct((NROWS, E), jnp.float32),
    mesh=_mesh,
    scratch_types=[
        pltpu.VMEM_SHARED((1024, E), jnp.float32),
        pltpu.VMEM_SHARED((1024, E), jnp.float32),
        pltpu.VMEM_SHARED((1024, E), jnp.float32),
        pltpu.VMEM((A, NG, KSUB, GSUB), jnp.int32),
        pltpu.VMEM((2, G, E), jnp.float32),
        pltpu.SemaphoreType.DMA,
        pltpu.SemaphoreType.DMA,
    ],
)(_sc_body)


def kernel(position, table0, table1, table2):
    idx = position.reshape(NROWS, A).T.reshape(A, NW, NG, KSUB, GSUB)
    idx = idx.transpose(1, 0, 2, 3, 4)
    out = _call(table0, table1, table2, idx)
    return out.reshape(N, S, E)


# speedup vs baseline: 2.1875x; 1.0225x over previous
import functools

import jax
import jax.numpy as jnp
from jax import lax
from jax.experimental import pallas as pl
from jax.experimental.pallas import tpu as pltpu
from jax.experimental.pallas import tpu_sc as plsc

N, S, A = 1024, 200, 3
E = 128
NROWS = N * S
NC, NSUB = 2, 16
NW = NC * NSUB
ROWS_PER_W = NROWS // NW
GSUB = 128
KSUB = 1
G = GSUB * KSUB
NG = ROWS_PER_W // G


def _sc_body(t0, t1, t2, idx_hbm, out_hbm, ts0, ts1, ts2, idxv, buf, sem0, sem1):
    c = lax.axis_index("c")
    s = lax.axis_index("s")
    wid = s * NC + c
    @pl.when(s == 0)
    def _stage():
        pltpu.sync_copy(t0, ts0)
        pltpu.sync_copy(t1, ts1)
        pltpu.sync_copy(t2, ts2)

    plsc.subcore_barrier()
    pltpu.sync_copy(idx_hbm.at[:, wid], idxv)
    tabs = (ts0, ts1, ts2)
    sems = (sem0, sem1)

    def zero(p):
        z = jnp.zeros((16,), jnp.float32)

        def row(r, carry):
            for cc in range(E // 16):
                buf[p, r, pl.ds(cc * 16, 16)] = z
            return carry

        lax.fori_loop(0, G, row, 0)

    def issue(g, p):
        for a in range(A):
            for k in range(KSUB):
                pltpu.async_copy(
                    tabs[a].at[idxv.at[a, g, k]],
                    buf.at[p, pl.ds(k * GSUB, GSUB)],
                    sems[p],
                    add=True,
                )

    def wait(g, p):
        for a in range(A):
            for k in range(KSUB):
                pltpu.make_async_copy(
                    tabs[a].at[idxv.at[a, g, k]],
                    buf.at[p, pl.ds(k * GSUB, GSUB)],
                    sems[p],
                ).wait()

    def out(g, p):
        base = (wid * NG + g) * G
        pltpu.sync_copy(buf.at[p], out_hbm.at[pl.ds(base, G)])

    zero(0)
    zero(1)
    issue(0, 0)

    def pair(i, carry):
        g = 2 * i
        issue(g + 1, 1)
        wait(g, 0)
        out(g, 0)
        zero(0)
        issue(g + 2, 0)
        wait(g + 1, 1)
        out(g + 1, 1)
        zero(1)
        return carry

    lax.fori_loop(0, NG // 2 - 1, pair, 0)
    issue(NG - 1, 1)
    wait(NG - 2, 0)
    out(NG - 2, 0)
    wait(NG - 1, 1)
    out(NG - 1, 1)


_mesh = plsc.VectorSubcoreMesh(
    core_axis_name="c", subcore_axis_name="s", num_cores=NC, num_subcores=NSUB
)

_call = functools.partial(
    pl.kernel,
    out_type=jax.ShapeDtypeStruct((NROWS, E), jnp.float32),
    mesh=_mesh,
    scratch_types=[
        pltpu.VMEM_SHARED((1024, E), jnp.float32),
        pltpu.VMEM_SHARED((1024, E), jnp.float32),
        pltpu.VMEM_SHARED((1024, E), jnp.float32),
        pltpu.VMEM((A, NG, KSUB, GSUB), jnp.int32),
        pltpu.VMEM((2, G, E), jnp.float32),
        pltpu.SemaphoreType.DMA,
        pltpu.SemaphoreType.DMA,
    ],
)(_sc_body)


def kernel(position, table0, table1, table2):
    idx = position.reshape(NROWS, A).T.reshape(A, NW, NG, KSUB, GSUB)
    out = _call(table0, table1, table2, idx)
    return out.reshape(N, S, E)


# speedup vs baseline: 2.2348x; 1.0216x over previous
import functools

import jax
import jax.numpy as jnp
from jax import lax
from jax.experimental import pallas as pl
from jax.experimental.pallas import tpu as pltpu
from jax.experimental.pallas import tpu_sc as plsc

N, S, A = 1024, 200, 3
E = 128
NROWS = N * S
NC, NSUB = 2, 16
NW = NC * NSUB
ROWS_PER_W = NROWS // NW
GSUB = 128
KSUB = 1
G = GSUB * KSUB
NG = ROWS_PER_W // G


def _sc_body(t0, t1, t2, idx_hbm, out_hbm, ts0, ts1, ts2, idxv, buf, sem0, sem1, sem2):
    c = lax.axis_index("c")
    s = lax.axis_index("s")
    wid = s * NC + c
    @pl.when(s == 0)
    def _stage():
        pltpu.sync_copy(t0, ts0)
        pltpu.sync_copy(t1, ts1)
        pltpu.sync_copy(t2, ts2)

    plsc.subcore_barrier()
    pltpu.sync_copy(idx_hbm.at[:, wid], idxv)
    tabs = (ts0, ts1, ts2)
    sems = (sem0, sem1, sem2)

    def zero(p):
        z = jnp.zeros((16,), jnp.float32)

        def row(r, carry):
            for cc in range(E // 16):
                buf[p, r, pl.ds(cc * 16, 16)] = z
            return carry

        lax.fori_loop(0, G, row, 0)

    def issue(g, p):
        for a in range(A):
            for k in range(KSUB):
                pltpu.async_copy(
                    tabs[a].at[idxv.at[a, g, k]],
                    buf.at[p, pl.ds(k * GSUB, GSUB)],
                    sems[p],
                    add=True,
                )

    def wait(g, p):
        for a in range(A):
            for k in range(KSUB):
                pltpu.make_async_copy(
                    tabs[a].at[idxv.at[a, g, k]],
                    buf.at[p, pl.ds(k * GSUB, GSUB)],
                    sems[p],
                ).wait()

    def out(g, p):
        base = (wid * NG + g) * G
        pltpu.sync_copy(buf.at[p], out_hbm.at[pl.ds(base, G)])

    zero(0)
    zero(1)
    issue(0, 0)
    issue(1, 1)

    def trip(i, carry):
        g0 = 3 * i
        for k in range(3):
            g = g0 + k
            q = (k + 2) % 3
            zero(q)
            issue(g + 2, q)
            wait(g, k)
            out(g, k)
        return carry

    lax.fori_loop(0, (NG - 2) // 3, trip, 0)
    wait(NG - 2, (NG - 2) % 3)
    out(NG - 2, (NG - 2) % 3)
    wait(NG - 1, (NG - 1) % 3)
    out(NG - 1, (NG - 1) % 3)


_mesh = plsc.VectorSubcoreMesh(
    core_axis_name="c", subcore_axis_name="s", num_cores=NC, num_subcores=NSUB
)

_call = functools.partial(
    pl.kernel,
    out_type=jax.ShapeDtypeStruct((NROWS, E), jnp.float32),
    mesh=_mesh,
    scratch_types=[
        pltpu.VMEM_SHARED((1024, E), jnp.float32),
        pltpu.VMEM_SHARED((1024, E), jnp.float32),
        pltpu.VMEM_SHARED((1024, E), jnp.float32),
        pltpu.VMEM((A, NG, KSUB, GSUB), jnp.int32),
        pltpu.VMEM((3, G, E), jnp.float32),
        pltpu.SemaphoreType.DMA,
        pltpu.SemaphoreType.DMA,
        pltpu.SemaphoreType.DMA,
    ],
)(_sc_body)


def kernel(position, table0, table1, table2):
    idx = position.reshape(NROWS, A).T.reshape(A, NW, NG, KSUB, GSUB)
    out = _call(table0, table1, table2, idx)
    return out.reshape(N, S, E)


# speedup vs baseline: 2.2632x; 1.0127x over previous
import functools

import jax
import jax.numpy as jnp
from jax import lax
from jax.experimental import pallas as pl
from jax.experimental.pallas import tpu as pltpu
from jax.experimental.pallas import tpu_sc as plsc

N, S, A = 1024, 200, 3
E = 128
NROWS = N * S
NC, NSUB = 2, 16
NW = NC * NSUB
ROWS_PER_W = NROWS // NW
GSUB = 128
KSUB = 1
G = GSUB * KSUB
NG = ROWS_PER_W // G


def _sc_body(t0, t1, t2, idx_hbm, out_hbm, ts0, ts1, ts2, idxv, buf, sem0, sem1, sem2):
    c = lax.axis_index("c")
    s = lax.axis_index("s")
    wid = s * NC + c
    @pl.when(s == 0)
    def _stage():
        pltpu.sync_copy(t0, ts0)
        pltpu.sync_copy(t1, ts1)
        pltpu.sync_copy(t2, ts2)

    plsc.subcore_barrier()
    pltpu.sync_copy(idx_hbm.at[:, wid], idxv)
    tabs = (ts0, ts1, ts2)
    sems = (sem0, sem1, sem2)

    def issue_first(g, p):
        pltpu.async_copy(
            tabs[0].at[idxv.at[0, g, 0]], buf.at[p], sems[p]
        )

    def issue_rest(g, p):
        for a in (1, 2):
            pltpu.async_copy(
                tabs[a].at[idxv.at[a, g, 0]], buf.at[p], sems[p], add=True
            )

    def wait_first(g, p):
        pltpu.make_async_copy(
            tabs[0].at[idxv.at[0, g, 0]], buf.at[p], sems[p]
        ).wait()

    def wait_rest(g, p):
        for a in (1, 2):
            pltpu.make_async_copy(
                tabs[a].at[idxv.at[a, g, 0]], buf.at[p], sems[p]
            ).wait()

    def out(g, p):
        base = (wid * NG + g) * G
        pltpu.sync_copy(buf.at[p], out_hbm.at[pl.ds(base, G)])

    issue_first(0, 0)
    issue_first(1, 1)
    wait_first(0, 0)
    issue_rest(0, 0)

    def step(g, k):
        issue_first(g + 2, (k + 2) % 3)
        wait_first(g + 1, (k + 1) % 3)
        issue_rest(g + 1, (k + 1) % 3)
        wait_rest(g, k)
        out(g, k)

    def trip(i, carry):
        g0 = 3 * i
        for k in range(3):
            step(g0 + k, k)
        return carry

    lax.fori_loop(0, (NG - 2) // 3, trip, 0)
    wait_first(NG - 1, (NG - 1) % 3)
    issue_rest(NG - 1, (NG - 1) % 3)
    wait_rest(NG - 2, (NG - 2) % 3)
    out(NG - 2, (NG - 2) % 3)
    wait_rest(NG - 1, (NG - 1) % 3)
    out(NG - 1, (NG - 1) % 3)


_mesh = plsc.VectorSubcoreMesh(
    core_axis_name="c", subcore_axis_name="s", num_cores=NC, num_subcores=NSUB
)

_call = functools.partial(
    pl.kernel,
    out_type=jax.ShapeDtypeStruct((NROWS, E), jnp.float32),
    mesh=_mesh,
    scratch_types=[
        pltpu.VMEM_SHARED((1024, E), jnp.float32),
        pltpu.VMEM_SHARED((1024, E), jnp.float32),
        pltpu.VMEM_SHARED((1024, E), jnp.float32),
        pltpu.VMEM((A, NG, KSUB, GSUB), jnp.int32),
        pltpu.VMEM((3, G, E), jnp.float32),
        pltpu.SemaphoreType.DMA,
        pltpu.SemaphoreType.DMA,
        pltpu.SemaphoreType.DMA,
    ],
)(_sc_body)


def kernel(position, table0, table1, table2):
    idx = position.reshape(NROWS, A).T.reshape(A, NW, NG, KSUB, GSUB)
    out = _call(table0, table1, table2, idx)
    return out.reshape(N, S, E)


# speedup vs baseline: 2.2908x; 1.0122x over previous
import functools

import jax
import jax.numpy as jnp
from jax import lax
from jax.experimental import pallas as pl
from jax.experimental.pallas import tpu as pltpu
from jax.experimental.pallas import tpu_sc as plsc

N, S, A = 1024, 200, 3
E = 128
NROWS = N * S
NC, NSUB = 2, 16
NW = NC * NSUB
ROWS_PER_W = NROWS // NW
GSUB = 128
KSUB = 1
G = GSUB * KSUB
NG = ROWS_PER_W // G


def _sc_body(t0, t1, t2, idx0_hbm, idx1_hbm, idx2_hbm, out_hbm, ts0, ts1, ts2, idxv, buf, sem0, sem1, sem2):
    c = lax.axis_index("c")
    s = lax.axis_index("s")
    wid = s * NC + c
    @pl.when(s == 0)
    def _stage():
        pltpu.sync_copy(t0, ts0)
        pltpu.sync_copy(t1, ts1)
        pltpu.sync_copy(t2, ts2)

    plsc.subcore_barrier()
    pltpu.sync_copy(idx0_hbm.at[wid], idxv.at[0])
    pltpu.sync_copy(idx1_hbm.at[wid], idxv.at[1])
    pltpu.sync_copy(idx2_hbm.at[wid], idxv.at[2])
    tabs = (ts0, ts1, ts2)
    sems = (sem0, sem1, sem2)

    def issue_first(g, p):
        pltpu.async_copy(
            tabs[0].at[idxv.at[0, g, 0]], buf.at[p], sems[p]
        )

    def issue_rest(g, p):
        for a in (1, 2):
            pltpu.async_copy(
                tabs[a].at[idxv.at[a, g, 0]], buf.at[p], sems[p], add=True
            )

    def wait_first(g, p):
        pltpu.make_async_copy(
            tabs[0].at[idxv.at[0, g, 0]], buf.at[p], sems[p]
        ).wait()

    def wait_rest(g, p):
        for a in (1, 2):
            pltpu.make_async_copy(
                tabs[a].at[idxv.at[a, g, 0]], buf.at[p], sems[p]
            ).wait()

    def out(g, p):
        base = (wid * NG + g) * G
        pltpu.sync_copy(buf.at[p], out_hbm.at[pl.ds(base, G)])

    issue_first(0, 0)
    issue_first(1, 1)
    wait_first(0, 0)
    issue_rest(0, 0)

    def step(g, k):
        issue_first(g + 2, (k + 2) % 3)
        wait_first(g + 1, (k + 1) % 3)
        issue_rest(g + 1, (k + 1) % 3)
        wait_rest(g, k)
        out(g, k)

    def trip(i, carry):
        g0 = 3 * i
        for k in range(3):
            step(g0 + k, k)
        return carry

    lax.fori_loop(0, (NG - 2) // 3, trip, 0)
    wait_first(NG - 1, (NG - 1) % 3)
    issue_rest(NG - 1, (NG - 1) % 3)
    wait_rest(NG - 2, (NG - 2) % 3)
    out(NG - 2, (NG - 2) % 3)
    wait_rest(NG - 1, (NG - 1) % 3)
    out(NG - 1, (NG - 1) % 3)


_mesh = plsc.VectorSubcoreMesh(
    core_axis_name="c", subcore_axis_name="s", num_cores=NC, num_subcores=NSUB
)

_call = functools.partial(
    pl.kernel,
    out_type=jax.ShapeDtypeStruct((NROWS, E), jnp.float32),
    mesh=_mesh,
    scratch_types=[
        pltpu.VMEM_SHARED((1024, E), jnp.float32),
        pltpu.VMEM_SHARED((1024, E), jnp.float32),
        pltpu.VMEM_SHARED((1024, E), jnp.float32),
        pltpu.VMEM((A, NG, KSUB, GSUB), jnp.int32),
        pltpu.VMEM((3, G, E), jnp.float32),
        pltpu.SemaphoreType.DMA,
        pltpu.SemaphoreType.DMA,
        pltpu.SemaphoreType.DMA,
    ],
)(_sc_body)


def kernel(position, table0, table1, table2):
    idxs = [position[:, :, a].reshape(NW, NG, KSUB, GSUB) for a in range(A)]
    out = _call(table0, table1, table2, *idxs)
    return out.reshape(N, S, E)
